# Initial kernel scaffold; baseline (speedup 1.0000x reference)
import jax
import jax.numpy as jnp
from jax.experimental import pallas as pl


def kernel(x, edge_index, edge_attr, W_in, b_in, W_edge, b_edge, root, bias, W_out, b_out):
    raise NotImplementedError("write your pallas kernel here")


# speedup vs baseline: 3.1281x; 3.1281x over previous
import functools

import jax
import jax.numpy as jnp
from jax import lax
from jax.experimental import pallas as pl
from jax.experimental.pallas import tpu as pltpu
from jax.experimental.pallas import tpu_sc as plsc

_N = 10000
_E = 320000
_DIN = 128
_CW = 16

_NC = 2
_NS = 16
_NW = _NC * _NS
_PER_TILE = _E // _NW
_CH = 80
_NCH = _PER_TILE // _CH
_ROWS_PER_TILE = _N // _NS

_LEAK = 0.01


def _lrelu(v):
    return jnp.where(v > 0, v, _LEAK * v)


def _h_body(x_ref, win_ref, bin_ref, h_ref):
    h = jnp.dot(x_ref[...], win_ref[...], preferred_element_type=jnp.float32)
    h_ref[...] = _lrelu(h + bin_ref[...])


def _compute_h(x, W_in, b_in):
    bn = 1000
    return pl.pallas_call(
        _h_body,
        grid=(_N // bn,),
        in_specs=[
            pl.BlockSpec((bn, _DIN), lambda i: (i, 0)),
            pl.BlockSpec((_DIN, _CW), lambda i: (0, 0)),
            pl.BlockSpec((1, _CW), lambda i: (0, 0)),
        ],
        out_specs=pl.BlockSpec((bn, _CW), lambda i: (i, 0)),
        out_shape=jax.ShapeDtypeStruct((_N, _CW), jnp.float32),
    )(x, W_in, b_in.reshape(1, _CW))


def _gather_body(h_hbm, src_hbm, out_hbm, idx_v, rows_v, sem):
    c = lax.axis_index("c")
    s = lax.axis_index("s")
    wid = s * _NC + c
    base = wid * _PER_TILE
    pltpu.sync_copy(src_hbm.at[wid], idx_v)

    def body(r, carry):
        pltpu.async_copy(h_hbm.at[idx_v.at[r]], rows_v, sem).wait()
        pltpu.sync_copy(rows_v, out_hbm.at[pl.ds(base + r * _CH, _CH)])
        return carry

    lax.fori_loop(0, _NCH, body, 0)


def _gather_rows(h, src2):
    mesh = plsc.VectorSubcoreMesh(core_axis_name="c", subcore_axis_name="s")
    return pl.kernel(
        _gather_body,
        out_type=jax.ShapeDtypeStruct((_E, _CW), jnp.float32),
        mesh=mesh,
        scratch_types=[
            pltpu.VMEM((_NCH, _CH), jnp.int32),
            pltpu.VMEM((_CH, _CW), jnp.float32),
            pltpu.SemaphoreType.DMA,
        ],
        compiler_params=pltpu.CompilerParams(use_tc_tiling_on_sc=False),
    )(h, src2)


def _msg_body(ea_ref, xj_ref, we_ref, be_ref, rep_ref, s_ref, msg_ref):
    z = jnp.dot(ea_ref[...], we_ref[...], preferred_element_type=jnp.float32)
    w = _lrelu(z + be_ref[...])
    xb = jnp.dot(xj_ref[...], rep_ref[...], preferred_element_type=jnp.float32)
    msg_ref[...] = jnp.dot(w * xb, s_ref[...], preferred_element_type=jnp.float32)


def _compute_msg(edge_attr, x_j, W_edge, b_edge):
    be = 2000
    rep = jnp.repeat(jnp.eye(_CW, dtype=jnp.float32), _CW, axis=1)
    smat = jnp.tile(jnp.eye(_CW, dtype=jnp.float32), (_CW, 1))
    return pl.pallas_call(
        _msg_body,
        grid=(_E // be,),
        in_specs=[
            pl.BlockSpec((be, 4), lambda i: (i, 0)),
            pl.BlockSpec((be, _CW), lambda i: (i, 0)),
            pl.BlockSpec((4, _CW * _CW), lambda i: (0, 0)),
            pl.BlockSpec((1, _CW * _CW), lambda i: (0, 0)),
            pl.BlockSpec((_CW, _CW * _CW), lambda i: (0, 0)),
            pl.BlockSpec((_CW * _CW, _CW), lambda i: (0, 0)),
        ],
        out_specs=pl.BlockSpec((be, _CW), lambda i: (i, 0)),
        out_shape=jax.ShapeDtypeStruct((_E, _CW), jnp.float32),
    )(edge_attr, x_j, W_edge, b_edge.reshape(1, _CW * _CW), rep, smat)


def _scatter_body(msg_hbm, dst_hbm, zer_hbm, out_hbm, idx_v, msg_v, aggr_sh):
    c = lax.axis_index("c")
    s = lax.axis_index("s")
    wid = s * _NC + c
    base = wid * _PER_TILE
    pltpu.sync_copy(dst_hbm.at[wid], idx_v)
    row0 = s * _ROWS_PER_TILE
    pltpu.sync_copy(zer_hbm.at[pl.ds(row0, _ROWS_PER_TILE)],
                    aggr_sh.at[pl.ds(row0, _ROWS_PER_TILE)])
    plsc.subcore_barrier()

    def body(r, carry):
        pltpu.sync_copy(msg_hbm.at[pl.ds(base + r * _CH, _CH)], msg_v)
        pltpu.sync_copy(msg_v, aggr_sh.at[idx_v.at[r]], add=True)
        return carry

    lax.fori_loop(0, _NCH, body, 0)
    plsc.subcore_barrier()
    pltpu.sync_copy(aggr_sh.at[pl.ds(row0, _ROWS_PER_TILE)],
                    out_hbm.at[c, pl.ds(row0, _ROWS_PER_TILE)])


def _scatter_add(msg, dst2, zeros_n):
    mesh = plsc.VectorSubcoreMesh(core_axis_name="c", subcore_axis_name="s")
    return pl.kernel(
        _scatter_body,
        out_type=jax.ShapeDtypeStruct((_NC, _N, _CW), jnp.float32),
        mesh=mesh,
        scratch_types=[
            pltpu.VMEM((_NCH, _CH), jnp.int32),
            pltpu.VMEM((_CH, _CW), jnp.float32),
            pltpu.VMEM_SHARED((_N, _CW), jnp.float32),
        ],
        compiler_params=pltpu.CompilerParams(use_tc_tiling_on_sc=False),
    )(msg, dst2, zeros_n)


def _out_body(aggr_ref, h_ref, root_ref, bias_ref, wout_ref, bout_ref, o_ref):
    a = aggr_ref[0] + aggr_ref[1]
    a = a + jnp.dot(h_ref[...], root_ref[...], preferred_element_type=jnp.float32)
    a = _lrelu(a + bias_ref[...])
    o_ref[...] = jnp.dot(a, wout_ref[...], preferred_element_type=jnp.float32) + bout_ref[...]


def _compute_out(aggr2, h, root, bias, W_out, b_out):
    bn = 2000
    return pl.pallas_call(
        _out_body,
        grid=(_N // bn,),
        in_specs=[
            pl.BlockSpec((_NC, bn, _CW), lambda i: (0, i, 0)),
            pl.BlockSpec((bn, _CW), lambda i: (i, 0)),
            pl.BlockSpec((_CW, _CW), lambda i: (0, 0)),
            pl.BlockSpec((1, _CW), lambda i: (0, 0)),
            pl.BlockSpec((_CW, 1), lambda i: (0, 0)),
            pl.BlockSpec((1, 1), lambda i: (0, 0)),
        ],
        out_specs=pl.BlockSpec((bn, 1), lambda i: (i, 0)),
        out_shape=jax.ShapeDtypeStruct((_N, 1), jnp.float32),
    )(aggr2, h, root, bias.reshape(1, _CW), W_out, b_out.reshape(1, 1))


def kernel(x, edge_index, edge_attr, W_in, b_in, W_edge, b_edge, root, bias, W_out, b_out):
    src2 = edge_index[0].reshape(_NW, _NCH, _CH)
    dst2 = edge_index[1].reshape(_NW, _NCH, _CH)
    h = _compute_h(x, W_in, b_in)
    x_j = _gather_rows(h, src2)
    msg = _compute_msg(edge_attr, x_j, W_edge, b_edge)
    zeros_n = jnp.zeros((_N, _CW), dtype=jnp.float32)
    aggr2 = _scatter_add(msg, dst2, zeros_n)
    return _compute_out(aggr2, h, root, bias, W_out, b_out)
